# baseline (speedup 1.0000x reference)
import functools
import math

import jax
import jax.numpy as jnp
from jax import lax
from jax.experimental import pallas as pl
from jax.experimental.pallas import tpu as pltpu
from jax.experimental.pallas import tpu_sc as plsc

N = 50000
E = 800000
EP = 819200
L = 6
H = 64
NF = 64
NG = 50
OUT = 32
IN_DIM = 28
CUTOFF = 6.0
NGRAPH = 64
NPAD = 50008
LOG2 = math.log(2.0)

NB = 1000
NNODE_BLK = N // NB
EB = 1024
NEDGE_BLK = EP // EB

EPT = EP // 16
ECH = 256
NCH = EPT // ECH
AGGR = 50048
ROWS_PT = AGGR // 16
DRB = 136
NDR = ROWS_PT // DRB

D2_PW = EP // 32
D2_HALF = D2_PW // 2
D2_CH = 6400


def _sp(a):
    return jnp.maximum(a, 0.0) + jnp.log(1.0 + jnp.exp(-jnp.abs(a))) - LOG2


def _d2_body(posx, posy, posz, row1, col1, d2_out, posc, rowb, colb, d2b):
    c = lax.axis_index("c")
    s = lax.axis_index("s")
    wid = s * 2 + c
    base = wid * D2_PW
    post = (posx, posy, posz)
    for half in range(2):
        hbase = base + half * D2_HALF
        for coord in range(3):
            pltpu.sync_copy(post[coord], posc)
            for ch in range(2):
                cbase = hbase + ch * D2_CH
                pltpu.sync_copy(row1.at[pl.ds(cbase, D2_CH)], rowb)
                pltpu.sync_copy(col1.at[pl.ds(cbase, D2_CH)], colb)
                off = ch * D2_CH

                def go(j, _, off=off, first=(coord == 0)):
                    ri = rowb[pl.ds(j * 16, 16)]
                    ci = colb[pl.ds(j * 16, 16)]
                    a = plsc.load_gather(posc, [ri])
                    b = plsc.load_gather(posc, [ci])
                    dd = a - b
                    sl = pl.ds(off + j * 16, 16)
                    if first:
                        d2b[sl] = dd * dd
                    else:
                        d2b[sl] = d2b[sl] + dd * dd
                    return _

                lax.fori_loop(0, D2_CH // 16, go, 0)
        pltpu.sync_copy(d2b, d2_out.at[pl.ds(hbase, D2_HALF)])


def _d2_call(posx, posy, posz, row_p, col_p):
    f = pl.kernel(
        _d2_body,
        out_type=jax.ShapeDtypeStruct((EP,), jnp.float32),
        mesh=plsc.VectorSubcoreMesh(core_axis_name="c", subcore_axis_name="s"),
        scratch_types=[
            pltpu.VMEM((NPAD,), jnp.float32),
            pltpu.VMEM((D2_CH,), jnp.int32),
            pltpu.VMEM((D2_CH,), jnp.int32),
            pltpu.VMEM((D2_HALF,), jnp.float32),
        ],
        compiler_params=pltpu.CompilerParams(needs_layout_passes=False),
    )
    return f(posx, posy, posz, row_p, col_p)


def _edge_body_full(vlh, wf, row1, col1, agg_out,
                    rowv, colv, rows, wfv, zbuf, agg_spm, gsem):
    c = lax.axis_index("c")
    s = lax.axis_index("s")

    def zb(m, _):
        zbuf[m, pl.ds(0, 16)] = jnp.zeros((16,), jnp.float32)
        zbuf[m, pl.ds(16, 16)] = jnp.zeros((16,), jnp.float32)
        return _

    lax.fori_loop(0, DRB, zb, 0)

    def zs(k, _):
        pltpu.sync_copy(zbuf, agg_spm.at[pl.ds(s * ROWS_PT + k * DRB, DRB)])
        return _

    lax.fori_loop(0, NDR, zs, 0)
    plsc.subcore_barrier()

    def chunk(i, _):
        e0 = s * EPT + i * ECH
        for j in range(ECH // 128):
            pltpu.sync_copy(row1.at[pl.ds(e0 + j * 128, 128)], rowv.at[j])
            pltpu.sync_copy(col1.at[pl.ds(e0 + j * 128, 128)], colv.at[j])
        for j in range(ECH // 128):
            pltpu.async_copy(vlh.at[c].at[rowv.at[j]],
                             rows.at[pl.ds(j * 128, 128)], gsem)
        pltpu.sync_copy(wf.at[c].at[pl.ds(e0, ECH)], wfv)
        for j in range(ECH // 128):
            pltpu.make_async_copy(vlh.at[c].at[rowv.at[j]],
                                  rows.at[pl.ds(j * 128, 128)], gsem).wait()

        @plsc.parallel_loop(0, ECH, step=1, unroll=8)
        def _mul(m):
            rows[m, pl.ds(0, 16)] = rows[m, pl.ds(0, 16)] * wfv[m, pl.ds(0, 16)]
            rows[m, pl.ds(16, 16)] = rows[m, pl.ds(16, 16)] * wfv[m, pl.ds(16, 16)]

        for j in range(ECH // 128):
            pltpu.sync_copy(rows.at[pl.ds(j * 128, 128)],
                            agg_spm.at[colv.at[j]], add=True)
        return 0

    lax.fori_loop(0, NCH, chunk, 0)
    plsc.subcore_barrier()

    def drain(k, _):
        rbase = s * ROWS_PT + k * DRB
        pltpu.sync_copy(agg_spm.at[pl.ds(rbase, DRB)], zbuf)
        pltpu.sync_copy(zbuf, agg_out.at[c].at[pl.ds(rbase, DRB)])
        return _

    lax.fori_loop(0, NDR, drain, 0)


def _edge_call(vlh, wf, row1, col1):
    f = pl.kernel(
        _edge_body_full,
        out_type=jax.ShapeDtypeStruct((2, AGGR, OUT), jnp.float32),
        mesh=plsc.VectorSubcoreMesh(core_axis_name="c", subcore_axis_name="s"),
        scratch_types=[
            pltpu.VMEM((ECH // 128, 128), jnp.int32),
            pltpu.VMEM((ECH // 128, 128), jnp.int32),
            pltpu.VMEM((ECH, OUT), jnp.float32),
            pltpu.VMEM((ECH, OUT), jnp.float32),
            pltpu.VMEM((DRB, OUT), jnp.float32),
            pltpu.VMEM_SHARED((AGGR, OUT), jnp.float32),
            pltpu.SemaphoreType.DMA,
        ],
        compiler_params=pltpu.CompilerParams(needs_layout_passes=False,
                                             use_tc_tiling_on_sc=False),
    )
    return f(vlh, wf, row1, col1)


def _wf_body(d2_ref, w1_ref, b1_ref, w2_ref, b2_ref, out_ref):
    step = CUTOFF / (NG - 1)
    coeff = -0.5 / (step * step)
    d2 = d2_ref[...]
    dist = jnp.sqrt(d2)
    ioff = lax.broadcasted_iota(jnp.int32, (EB, NG), 1).astype(jnp.float32) * step
    emb = jnp.exp(coeff * (dist - ioff) ** 2)
    cutf = 0.5 * (jnp.cos(dist * (math.pi / CUTOFF)) + 1.0)
    h1 = _sp(jnp.dot(emb, w1_ref[...], preferred_element_type=jnp.float32)
             + b1_ref[...])
    wfl = jnp.dot(h1, w2_ref[...], preferred_element_type=jnp.float32) \
        + b2_ref[...]
    wfl = wfl * cutf
    out_ref[0] = wfl[:, :OUT]
    out_ref[1] = wfl[:, OUT:]


def _wf_call(d2c, w1, b1, w2, b2):
    return pl.pallas_call(
        _wf_body,
        grid=(NEDGE_BLK,),
        in_specs=[
            pl.BlockSpec((EB, 1), lambda i: (i, 0)),
            pl.BlockSpec((NG, NF), lambda i: (0, 0)),
            pl.BlockSpec((1, NF), lambda i: (0, 0)),
            pl.BlockSpec((NF, NF), lambda i: (0, 0)),
            pl.BlockSpec((1, NF), lambda i: (0, 0)),
        ],
        out_specs=pl.BlockSpec((2, EB, OUT), lambda i: (0, i, 0)),
        out_shape=jax.ShapeDtypeStruct((2, EP, OUT), jnp.float32),
    )(d2c, w1, b1, w2, b2)


def _fe1_body(x_ref, w_ref, b_ref, h_ref, st_ref, acc_ref):
    i = pl.program_id(0)
    h = jnp.dot(x_ref[...], w_ref[...], preferred_element_type=jnp.float32) \
        + b_ref[...]
    h_ref[...] = h
    s1 = jnp.sum(h, axis=0, keepdims=True)
    s2 = jnp.sum(h * h, axis=0, keepdims=True)
    st = jnp.concatenate([s1, s2, jnp.zeros((6, H // 2), jnp.float32)], axis=0)

    @pl.when(i == 0)
    def _():
        acc_ref[...] = jnp.zeros_like(acc_ref)

    acc_ref[...] += st

    @pl.when(i == NNODE_BLK - 1)
    def _():
        st_ref[...] = acc_ref[...]


def _fe1_call(x, w1, b1):
    return pl.pallas_call(
        _fe1_body,
        grid=(NNODE_BLK,),
        in_specs=[
            pl.BlockSpec((NB, IN_DIM), lambda i: (i, 0)),
            pl.BlockSpec((IN_DIM, H // 2), lambda i: (0, 0)),
            pl.BlockSpec((1, H // 2), lambda i: (0, 0)),
        ],
        out_specs=[
            pl.BlockSpec((NB, H // 2), lambda i: (i, 0)),
            pl.BlockSpec((8, H // 2), lambda i: (0, 0)),
        ],
        out_shape=[
            jax.ShapeDtypeStruct((N, H // 2), jnp.float32),
            jax.ShapeDtypeStruct((8, H // 2), jnp.float32),
        ],
        scratch_shapes=[pltpu.VMEM((8, H // 2), jnp.float32)],
    )(x, w1, b1)


def _fe2_body(h_ref, st_ref, g_ref, be_ref, w2_ref, b2_ref, elw_ref,
              v_ref, vlh_ref):
    st = st_ref[...]
    mean = st[0:1] / N
    var = st[1:2] / N - mean * mean
    inv = lax.rsqrt(var + 1e-5)
    hn = (h_ref[...] - mean) * inv * g_ref[...] + be_ref[...]
    hn = jnp.maximum(hn, 0.0)
    v = jnp.maximum(
        jnp.dot(hn, w2_ref[...], preferred_element_type=jnp.float32)
        + b2_ref[...], 0.0)
    v_ref[...] = v
    vl = jnp.dot(v, elw_ref[...], preferred_element_type=jnp.float32)
    vlh_ref[0] = vl[:, :OUT]
    vlh_ref[1] = vl[:, OUT:]


def _fe2_call(h, st, g, be, w2, b2, elw):
    return pl.pallas_call(
        _fe2_body,
        grid=(NNODE_BLK,),
        in_specs=[
            pl.BlockSpec((NB, H // 2), lambda i: (i, 0)),
            pl.BlockSpec((8, H // 2), lambda i: (0, 0)),
            pl.BlockSpec((1, H // 2), lambda i: (0, 0)),
            pl.BlockSpec((1, H // 2), lambda i: (0, 0)),
            pl.BlockSpec((H // 2, H), lambda i: (0, 0)),
            pl.BlockSpec((1, H), lambda i: (0, 0)),
            pl.BlockSpec((H, NF), lambda i: (0, 0)),
        ],
        out_specs=[
            pl.BlockSpec((NB, H), lambda i: (i, 0)),
            pl.BlockSpec((2, NB, OUT), lambda i: (0, i, 0)),
        ],
        out_shape=[
            jax.ShapeDtypeStruct((N, H), jnp.float32),
            jax.ShapeDtypeStruct((2, N, OUT), jnp.float32),
        ],
    )(h, st, g, be, w2, b2, elw)


def _upd_body(agg_ref, v_ref, w1_ref, b1_ref, w2_ref, b2_ref, elw_ref,
              vn_ref, vlh_ref):
    cat = jnp.concatenate([agg_ref[0], agg_ref[1]], axis=1)
    m = jnp.dot(_sp(jnp.dot(cat, w1_ref[...],
                            preferred_element_type=jnp.float32) + b1_ref[...]),
                w2_ref[...], preferred_element_type=jnp.float32) + b2_ref[...]
    vn = v_ref[...] + m
    vn_ref[...] = vn
    vl = jnp.dot(vn, elw_ref[...], preferred_element_type=jnp.float32)
    vlh_ref[0] = vl[:, :OUT]
    vlh_ref[1] = vl[:, OUT:]


def _upd_call(agg, v, w1, b1, w2, b2, elw):
    return pl.pallas_call(
        _upd_body,
        grid=(NNODE_BLK,),
        in_specs=[
            pl.BlockSpec((2, NB, OUT), lambda i: (0, i, 0)),
            pl.BlockSpec((NB, H), lambda i: (i, 0)),
            pl.BlockSpec((NF, H), lambda i: (0, 0)),
            pl.BlockSpec((1, H), lambda i: (0, 0)),
            pl.BlockSpec((H, H), lambda i: (0, 0)),
            pl.BlockSpec((1, H), lambda i: (0, 0)),
            pl.BlockSpec((H, NF), lambda i: (0, 0)),
        ],
        out_specs=[
            pl.BlockSpec((NB, H), lambda i: (i, 0)),
            pl.BlockSpec((2, NB, OUT), lambda i: (0, i, 0)),
        ],
        out_shape=[
            jax.ShapeDtypeStruct((N, H), jnp.float32),
            jax.ShapeDtypeStruct((2, N, OUT), jnp.float32),
        ],
    )(agg, v, w1, b1, w2, b2, elw)


def _out_body(agg_ref, v_ref, w1_ref, b1_ref, w2_ref, b2_ref,
              uw1_ref, ub1_ref, uw2_ref, ub2_ref, batch_ref,
              u_ref, acc_ref):
    i = pl.program_id(0)
    cat = jnp.concatenate([agg_ref[0], agg_ref[1]], axis=1)
    m = jnp.dot(_sp(jnp.dot(cat, w1_ref[...],
                            preferred_element_type=jnp.float32) + b1_ref[...]),
                w2_ref[...], preferred_element_type=jnp.float32) + b2_ref[...]
    vn = v_ref[...] + m
    hu = jnp.dot(_sp(jnp.dot(vn, uw1_ref[...],
                             preferred_element_type=jnp.float32) + ub1_ref[...]),
                 uw2_ref[...], preferred_element_type=jnp.float32) + ub2_ref[...]
    gid = lax.broadcasted_iota(jnp.int32, (NB, NGRAPH), 1)
    oh = (gid == batch_ref[...]).astype(jnp.float32)
    part = lax.dot_general(oh, hu, (((0,), (0,)), ((), ())),
                           preferred_element_type=jnp.float32)

    @pl.when(i == 0)
    def _():
        acc_ref[...] = jnp.zeros_like(acc_ref)

    acc_ref[...] += part

    @pl.when(i == NNODE_BLK - 1)
    def _():
        u_ref[...] = acc_ref[...]


def _out_call(agg, v, w1, b1, w2, b2, uw1, ub1, uw2, ub2, batch2):
    return pl.pallas_call(
        _out_body,
        grid=(NNODE_BLK,),
        in_specs=[
            pl.BlockSpec((2, NB, OUT), lambda i: (0, i, 0)),
            pl.BlockSpec((NB, H), lambda i: (i, 0)),
            pl.BlockSpec((NF, H), lambda i: (0, 0)),
            pl.BlockSpec((1, H), lambda i: (0, 0)),
            pl.BlockSpec((H, H), lambda i: (0, 0)),
            pl.BlockSpec((1, H), lambda i: (0, 0)),
            pl.BlockSpec((H, H // 2), lambda i: (0, 0)),
            pl.BlockSpec((1, H // 2), lambda i: (0, 0)),
            pl.BlockSpec((H // 2, OUT), lambda i: (0, 0)),
            pl.BlockSpec((1, OUT), lambda i: (0, 0)),
            pl.BlockSpec((NB, 1), lambda i: (i, 0)),
        ],
        out_specs=pl.BlockSpec((NGRAPH, OUT), lambda i: (0, 0)),
        out_shape=jax.ShapeDtypeStruct((NGRAPH, OUT), jnp.float32),
        scratch_shapes=[pltpu.VMEM((NGRAPH, OUT), jnp.float32)],
    )(agg, v, w1, b1, w2, b2, uw1, ub1, uw2, ub2, batch2)


def kernel(x, x_one_hot, pos, batch, edge_index,
           fe_w1, fe_b1, fe_gamma, fe_beta, fe_w2, fe_b2,
           e_lin_w, e_mlp_w1, e_mlp_b1, e_mlp_w2, e_mlp_b2,
           v_w1, v_b1, v_w2, v_b2, u_w1, u_b1, u_w2, u_b2):
    row = edge_index[0].astype(jnp.int32)
    col = edge_index[1].astype(jnp.int32)
    row_p = jnp.concatenate([row, jnp.zeros((EP - E,), jnp.int32)])
    col_p = jnp.concatenate([col, jnp.full((EP - E,), N, jnp.int32)])

    zpad = jnp.zeros((NPAD - N,), jnp.float32)
    posx = jnp.concatenate([pos[:, 0], zpad])
    posy = jnp.concatenate([pos[:, 1], zpad])
    posz = jnp.concatenate([pos[:, 2], zpad])

    d2 = _d2_call(posx, posy, posz, row_p, col_p)
    d2c = d2.reshape(EP, 1)

    wfs = [_wf_call(d2c, e_mlp_w1[l], e_mlp_b1[l].reshape(1, NF),
                    e_mlp_w2[l], e_mlp_b2[l].reshape(1, NF))
           for l in range(L)]

    h, st = _fe1_call(x, fe_w1, fe_b1.reshape(1, H // 2))
    v, vlh = _fe2_call(h, st, fe_gamma.reshape(1, H // 2),
                       fe_beta.reshape(1, H // 2), fe_w2,
                       fe_b2.reshape(1, H), e_lin_w[0])

    for l in range(L - 1):
        agg = _edge_call(vlh, wfs[l], row_p, col_p)
        v, vlh = _upd_call(agg, v, v_w1[l], v_b1[l].reshape(1, H),
                           v_w2[l], v_b2[l].reshape(1, H), e_lin_w[l + 1])

    agg = _edge_call(vlh, wfs[L - 1], row_p, col_p)
    batch2 = batch.astype(jnp.int32).reshape(N, 1)
    u = _out_call(agg, v, v_w1[L - 1], v_b1[L - 1].reshape(1, H),
                  v_w2[L - 1], v_b2[L - 1].reshape(1, H),
                  u_w1, u_b1.reshape(1, H // 2), u_w2, u_b2.reshape(1, OUT),
                  batch2)
    return u


# speedup vs baseline: 2.0297x; 2.0297x over previous
import functools
import math

import jax
import jax.numpy as jnp
from jax import lax
from jax.experimental import pallas as pl
from jax.experimental.pallas import tpu as pltpu
from jax.experimental.pallas import tpu_sc as plsc

N = 50000
E = 800000
EP = 819200
L = 6
H = 64
NF = 64
NG = 50
OUT = 32
IN_DIM = 28
CUTOFF = 6.0
NGRAPH = 64
NPAD = 50008
LOG2 = math.log(2.0)

NB = 1000
NNODE_BLK = N // NB
EB = 1024
NEDGE_BLK = EP // EB

EPT = EP // 16
ECH = 256
NCH = EPT // ECH
AGGR = 50048
ROWS_PT = AGGR // 16
DRB = 136
NDR = ROWS_PT // DRB

D2_PW = EP // 32
D2_HALF = D2_PW // 2
D2_CH = 6400


def _sp(a):
    return jnp.maximum(a, 0.0) + jnp.log(1.0 + jnp.exp(-jnp.abs(a))) - LOG2


def _d2_body(posx, posy, posz, row1, col1, d2_out, posc, rowb, colb, d2b):
    c = lax.axis_index("c")
    s = lax.axis_index("s")
    wid = s * 2 + c
    base = wid * D2_PW
    post = (posx, posy, posz)
    for half in range(2):
        hbase = base + half * D2_HALF
        for coord in range(3):
            pltpu.sync_copy(post[coord], posc)
            for ch in range(2):
                cbase = hbase + ch * D2_CH
                pltpu.sync_copy(row1.at[pl.ds(cbase, D2_CH)], rowb)
                pltpu.sync_copy(col1.at[pl.ds(cbase, D2_CH)], colb)
                off = ch * D2_CH

                def go(j, _, off=off, first=(coord == 0)):
                    ri = rowb[pl.ds(j * 16, 16)]
                    ci = colb[pl.ds(j * 16, 16)]
                    a = plsc.load_gather(posc, [ri])
                    b = plsc.load_gather(posc, [ci])
                    dd = a - b
                    sl = pl.ds(off + j * 16, 16)
                    if first:
                        d2b[sl] = dd * dd
                    else:
                        d2b[sl] = d2b[sl] + dd * dd
                    return _

                lax.fori_loop(0, D2_CH // 16, go, 0)
        pltpu.sync_copy(d2b, d2_out.at[pl.ds(hbase, D2_HALF)])


def _d2_call(posx, posy, posz, row_p, col_p):
    f = pl.kernel(
        _d2_body,
        out_type=jax.ShapeDtypeStruct((EP,), jnp.float32),
        mesh=plsc.VectorSubcoreMesh(core_axis_name="c", subcore_axis_name="s"),
        scratch_types=[
            pltpu.VMEM((NPAD,), jnp.float32),
            pltpu.VMEM((D2_CH,), jnp.int32),
            pltpu.VMEM((D2_CH,), jnp.int32),
            pltpu.VMEM((D2_HALF,), jnp.float32),
        ],
        compiler_params=pltpu.CompilerParams(needs_layout_passes=False),
    )
    return f(posx, posy, posz, row_p, col_p)


SUP = 2048
NSUP = EPT // SUP


def _edge_body_full(vlh, wf, row2, col2, agg_out,
                    rowv, colv, rows, wfv, agg_spm, gsem, wsem):
    c = lax.axis_index("c")
    s = lax.axis_index("s")

    @plsc.parallel_loop(0, DRB, step=1, unroll=8)
    def _z(m):
        rows[m, pl.ds(0, 16)] = jnp.zeros((16,), jnp.float32)
        rows[m, pl.ds(16, 16)] = jnp.zeros((16,), jnp.float32)

    def zs(k, _):
        pltpu.sync_copy(rows.at[pl.ds(0, DRB)],
                        agg_spm.at[pl.ds(s * ROWS_PT + k * DRB, DRB)])
        return 0

    lax.fori_loop(0, NDR, zs, 0)
    plsc.subcore_barrier()

    pltpu.async_copy(wf.at[c].at[pl.ds(s * EPT, ECH)], wfv.at[0], wsem)

    def sup(si, _):
        sb = s * EPT + si * SUP
        rb = s * (EPT // 128) + si * (SUP // 128)
        pltpu.sync_copy(row2.at[pl.ds(rb, SUP // 128)], rowv)
        pltpu.sync_copy(col2.at[pl.ds(rb, SUP // 128)], colv)
        for j in range(SUP // ECH):
            gi = si * (SUP // ECH) + j
            e0 = sb + j * ECH
            buf = j % 2
            for q in range(ECH // 128):
                pltpu.async_copy(vlh.at[c].at[rowv.at[(ECH // 128) * j + q]],
                                 rows.at[pl.ds(q * 128, 128)], gsem)
            e0n = s * EPT + jnp.minimum(gi + 1, NCH - 1) * ECH
            pltpu.async_copy(wf.at[c].at[pl.ds(e0n, ECH)],
                             wfv.at[1 - buf], wsem)
            pltpu.make_async_copy(wf.at[c].at[pl.ds(e0, ECH)],
                                  wfv.at[buf], wsem).wait()
            for q in range(ECH // 128):
                pltpu.make_async_copy(vlh.at[c].at[rowv.at[(ECH // 128) * j + q]],
                                      rows.at[pl.ds(q * 128, 128)], gsem).wait()

            @plsc.parallel_loop(0, ECH, step=1, unroll=8)
            def _mul(m, buf=buf):
                rows[m, pl.ds(0, 16)] = (rows[m, pl.ds(0, 16)]
                                         * wfv[buf, m, pl.ds(0, 16)])
                rows[m, pl.ds(16, 16)] = (rows[m, pl.ds(16, 16)]
                                          * wfv[buf, m, pl.ds(16, 16)])

            for q in range(ECH // 128):
                pltpu.sync_copy(rows.at[pl.ds(q * 128, 128)],
                                agg_spm.at[colv.at[(ECH // 128) * j + q]],
                                add=True)
        return 0

    lax.fori_loop(0, NSUP, sup, 0)
    pltpu.make_async_copy(wf.at[c].at[pl.ds(s * EPT, ECH)],
                          wfv.at[0], wsem).wait()
    plsc.subcore_barrier()

    def drain(k, _):
        rbase = s * ROWS_PT + k * DRB
        pltpu.sync_copy(agg_spm.at[pl.ds(rbase, DRB)], rows.at[pl.ds(0, DRB)])
        pltpu.sync_copy(rows.at[pl.ds(0, DRB)],
                        agg_out.at[c].at[pl.ds(rbase, DRB)])
        return 0

    lax.fori_loop(0, NDR, drain, 0)


def _edge_call(vlh, wf, row2, col2):
    f = pl.kernel(
        _edge_body_full,
        out_type=jax.ShapeDtypeStruct((2, AGGR, OUT), jnp.float32),
        mesh=plsc.VectorSubcoreMesh(core_axis_name="c", subcore_axis_name="s"),
        scratch_types=[
            pltpu.VMEM((SUP // 128, 128), jnp.int32),
            pltpu.VMEM((SUP // 128, 128), jnp.int32),
            pltpu.VMEM((ECH, OUT), jnp.float32),
            pltpu.VMEM((2, ECH, OUT), jnp.float32),
            pltpu.VMEM_SHARED((AGGR, OUT), jnp.float32),
            pltpu.SemaphoreType.DMA,
            pltpu.SemaphoreType.DMA,
        ],
        compiler_params=pltpu.CompilerParams(needs_layout_passes=False,
                                             use_tc_tiling_on_sc=False),
    )
    return f(vlh, wf, row2, col2)


def _wf_body(d2_ref, w1a_ref, b1a_ref, w2_ref, b2_ref, eye_ref, *out_refs):
    step = CUTOFF / (NG - 1)
    coeff = -0.5 / (step * step)
    d2 = d2_ref[...]
    dist = jnp.sqrt(d2)
    cutf = 0.5 * (jnp.cos(dist * (math.pi / CUTOFF)) + 1.0)
    offs = (lax.broadcasted_iota(jnp.int32, (NG, 128), 0).astype(jnp.float32)
            * step)
    eye = eye_ref[...]
    cols = []
    for g in range(8):
        dr = dist[g:g + 1, :]
        et = jnp.exp(coeff * (dr - offs) ** 2)
        etc = jnp.concatenate([et, cutf[g:g + 1, :]], axis=0)
        cols.append(lax.dot_general(eye, etc, (((1,), (1,)), ((), ())),
                                    preferred_element_type=jnp.float32))
    embc = jnp.concatenate(cols, axis=0)
    cc = embc[:, NG:NG + 1]
    h1 = _sp(jnp.dot(embc, w1a_ref[...],
                     preferred_element_type=jnp.float32) + b1a_ref[...])
    for l in range(L):
        wfl = jnp.dot(h1[:, l * NF:(l + 1) * NF], w2_ref[l],
                      preferred_element_type=jnp.float32) + b2_ref[l]
        wfl = wfl * cc
        out_refs[l][0] = wfl[:, :OUT]
        out_refs[l][1] = wfl[:, OUT:]


def _wf_call(d2m, w1a, b1a, w2, b2c, eye):
    return pl.pallas_call(
        _wf_body,
        grid=(NEDGE_BLK,),
        in_specs=[
            pl.BlockSpec((8, 128), lambda i: (i, 0)),
            pl.BlockSpec((NG + 1, L * NF), lambda i: (0, 0)),
            pl.BlockSpec((1, L * NF), lambda i: (0, 0)),
            pl.BlockSpec((L, NF, NF), lambda i: (0, 0, 0)),
            pl.BlockSpec((L, 1, NF), lambda i: (0, 0, 0)),
            pl.BlockSpec((128, 128), lambda i: (0, 0)),
        ],
        out_specs=[pl.BlockSpec((2, EB, OUT), lambda i: (0, i, 0))
                   for _ in range(L)],
        out_shape=[jax.ShapeDtypeStruct((2, EP, OUT), jnp.float32)
                   for _ in range(L)],
    )(d2m, w1a, b1a, w2, b2c, eye)


def _fe1_body(x_ref, w_ref, b_ref, h_ref, st_ref, acc_ref):
    i = pl.program_id(0)
    h = jnp.dot(x_ref[...], w_ref[...], preferred_element_type=jnp.float32) \
        + b_ref[...]
    h_ref[...] = h
    s1 = jnp.sum(h, axis=0, keepdims=True)
    s2 = jnp.sum(h * h, axis=0, keepdims=True)
    st = jnp.concatenate([s1, s2, jnp.zeros((6, H // 2), jnp.float32)], axis=0)

    @pl.when(i == 0)
    def _():
        acc_ref[...] = jnp.zeros_like(acc_ref)

    acc_ref[...] += st

    @pl.when(i == NNODE_BLK - 1)
    def _():
        st_ref[...] = acc_ref[...]


def _fe1_call(x, w1, b1):
    return pl.pallas_call(
        _fe1_body,
        grid=(NNODE_BLK,),
        in_specs=[
            pl.BlockSpec((NB, IN_DIM), lambda i: (i, 0)),
            pl.BlockSpec((IN_DIM, H // 2), lambda i: (0, 0)),
            pl.BlockSpec((1, H // 2), lambda i: (0, 0)),
        ],
        out_specs=[
            pl.BlockSpec((NB, H // 2), lambda i: (i, 0)),
            pl.BlockSpec((8, H // 2), lambda i: (0, 0)),
        ],
        out_shape=[
            jax.ShapeDtypeStruct((N, H // 2), jnp.float32),
            jax.ShapeDtypeStruct((8, H // 2), jnp.float32),
        ],
        scratch_shapes=[pltpu.VMEM((8, H // 2), jnp.float32)],
    )(x, w1, b1)


def _fe2_body(h_ref, st_ref, g_ref, be_ref, w2_ref, b2_ref, elw_ref,
              v_ref, vlh_ref):
    st = st_ref[...]
    mean = st[0:1] / N
    var = st[1:2] / N - mean * mean
    inv = lax.rsqrt(var + 1e-5)
    hn = (h_ref[...] - mean) * inv * g_ref[...] + be_ref[...]
    hn = jnp.maximum(hn, 0.0)
    v = jnp.maximum(
        jnp.dot(hn, w2_ref[...], preferred_element_type=jnp.float32)
        + b2_ref[...], 0.0)
    v_ref[...] = v
    vl = jnp.dot(v, elw_ref[...], preferred_element_type=jnp.float32)
    vlh_ref[0] = vl[:, :OUT]
    vlh_ref[1] = vl[:, OUT:]


def _fe2_call(h, st, g, be, w2, b2, elw):
    return pl.pallas_call(
        _fe2_body,
        grid=(NNODE_BLK,),
        in_specs=[
            pl.BlockSpec((NB, H // 2), lambda i: (i, 0)),
            pl.BlockSpec((8, H // 2), lambda i: (0, 0)),
            pl.BlockSpec((1, H // 2), lambda i: (0, 0)),
            pl.BlockSpec((1, H // 2), lambda i: (0, 0)),
            pl.BlockSpec((H // 2, H), lambda i: (0, 0)),
            pl.BlockSpec((1, H), lambda i: (0, 0)),
            pl.BlockSpec((H, NF), lambda i: (0, 0)),
        ],
        out_specs=[
            pl.BlockSpec((NB, H), lambda i: (i, 0)),
            pl.BlockSpec((2, NB, OUT), lambda i: (0, i, 0)),
        ],
        out_shape=[
            jax.ShapeDtypeStruct((N, H), jnp.float32),
            jax.ShapeDtypeStruct((2, N, OUT), jnp.float32),
        ],
    )(h, st, g, be, w2, b2, elw)


def _upd_body(agg_ref, v_ref, w1_ref, b1_ref, w2_ref, b2_ref, elw_ref,
              vn_ref, vlh_ref):
    cat = jnp.concatenate([agg_ref[0], agg_ref[1]], axis=1)
    m = jnp.dot(_sp(jnp.dot(cat, w1_ref[...],
                            preferred_element_type=jnp.float32) + b1_ref[...]),
                w2_ref[...], preferred_element_type=jnp.float32) + b2_ref[...]
    vn = v_ref[...] + m
    vn_ref[...] = vn
    vl = jnp.dot(vn, elw_ref[...], preferred_element_type=jnp.float32)
    vlh_ref[0] = vl[:, :OUT]
    vlh_ref[1] = vl[:, OUT:]


def _upd_call(agg, v, w1, b1, w2, b2, elw):
    return pl.pallas_call(
        _upd_body,
        grid=(NNODE_BLK,),
        in_specs=[
            pl.BlockSpec((2, NB, OUT), lambda i: (0, i, 0)),
            pl.BlockSpec((NB, H), lambda i: (i, 0)),
            pl.BlockSpec((NF, H), lambda i: (0, 0)),
            pl.BlockSpec((1, H), lambda i: (0, 0)),
            pl.BlockSpec((H, H), lambda i: (0, 0)),
            pl.BlockSpec((1, H), lambda i: (0, 0)),
            pl.BlockSpec((H, NF), lambda i: (0, 0)),
        ],
        out_specs=[
            pl.BlockSpec((NB, H), lambda i: (i, 0)),
            pl.BlockSpec((2, NB, OUT), lambda i: (0, i, 0)),
        ],
        out_shape=[
            jax.ShapeDtypeStruct((N, H), jnp.float32),
            jax.ShapeDtypeStruct((2, N, OUT), jnp.float32),
        ],
    )(agg, v, w1, b1, w2, b2, elw)


def _out_body(agg_ref, v_ref, w1_ref, b1_ref, w2_ref, b2_ref,
              uw1_ref, ub1_ref, uw2_ref, ub2_ref, batch_ref,
              u_ref, acc_ref):
    i = pl.program_id(0)
    cat = jnp.concatenate([agg_ref[0], agg_ref[1]], axis=1)
    m = jnp.dot(_sp(jnp.dot(cat, w1_ref[...],
                            preferred_element_type=jnp.float32) + b1_ref[...]),
                w2_ref[...], preferred_element_type=jnp.float32) + b2_ref[...]
    vn = v_ref[...] + m
    hu = jnp.dot(_sp(jnp.dot(vn, uw1_ref[...],
                             preferred_element_type=jnp.float32) + ub1_ref[...]),
                 uw2_ref[...], preferred_element_type=jnp.float32) + ub2_ref[...]
    gid = lax.broadcasted_iota(jnp.int32, (NB, NGRAPH), 1)
    oh = (gid == batch_ref[...]).astype(jnp.float32)
    part = lax.dot_general(oh, hu, (((0,), (0,)), ((), ())),
                           preferred_element_type=jnp.float32)

    @pl.when(i == 0)
    def _():
        acc_ref[...] = jnp.zeros_like(acc_ref)

    acc_ref[...] += part

    @pl.when(i == NNODE_BLK - 1)
    def _():
        u_ref[...] = acc_ref[...]


def _out_call(agg, v, w1, b1, w2, b2, uw1, ub1, uw2, ub2, batch2):
    return pl.pallas_call(
        _out_body,
        grid=(NNODE_BLK,),
        in_specs=[
            pl.BlockSpec((2, NB, OUT), lambda i: (0, i, 0)),
            pl.BlockSpec((NB, H), lambda i: (i, 0)),
            pl.BlockSpec((NF, H), lambda i: (0, 0)),
            pl.BlockSpec((1, H), lambda i: (0, 0)),
            pl.BlockSpec((H, H), lambda i: (0, 0)),
            pl.BlockSpec((1, H), lambda i: (0, 0)),
            pl.BlockSpec((H, H // 2), lambda i: (0, 0)),
            pl.BlockSpec((1, H // 2), lambda i: (0, 0)),
            pl.BlockSpec((H // 2, OUT), lambda i: (0, 0)),
            pl.BlockSpec((1, OUT), lambda i: (0, 0)),
            pl.BlockSpec((NB, 1), lambda i: (i, 0)),
        ],
        out_specs=pl.BlockSpec((NGRAPH, OUT), lambda i: (0, 0)),
        out_shape=jax.ShapeDtypeStruct((NGRAPH, OUT), jnp.float32),
        scratch_shapes=[pltpu.VMEM((NGRAPH, OUT), jnp.float32)],
    )(agg, v, w1, b1, w2, b2, uw1, ub1, uw2, ub2, batch2)


def kernel(x, x_one_hot, pos, batch, edge_index,
           fe_w1, fe_b1, fe_gamma, fe_beta, fe_w2, fe_b2,
           e_lin_w, e_mlp_w1, e_mlp_b1, e_mlp_w2, e_mlp_b2,
           v_w1, v_b1, v_w2, v_b2, u_w1, u_b1, u_w2, u_b2):
    row = edge_index[0].astype(jnp.int32)
    col = edge_index[1].astype(jnp.int32)
    row_p = jnp.concatenate([row, jnp.zeros((EP - E,), jnp.int32)])
    col_p = jnp.concatenate([col, jnp.full((EP - E,), N, jnp.int32)])
    row2 = row_p.reshape(EP // 128, 128)
    col2 = col_p.reshape(EP // 128, 128)

    zpad = jnp.zeros((NPAD - N,), jnp.float32)
    posx = jnp.concatenate([pos[:, 0], zpad])
    posy = jnp.concatenate([pos[:, 1], zpad])
    posz = jnp.concatenate([pos[:, 2], zpad])

    d2 = _d2_call(posx, posy, posz, row_p, col_p)
    d2m = d2.reshape(EP // 128, 128)

    w1a = jnp.concatenate([jnp.moveaxis(e_mlp_w1, 0, 1).reshape(NG, L * NF),
                           jnp.zeros((1, L * NF), jnp.float32)], axis=0)
    wfs = _wf_call(d2m, w1a, e_mlp_b1.reshape(1, L * NF),
                   e_mlp_w2, e_mlp_b2.reshape(L, 1, NF),
                   jnp.eye(128, dtype=jnp.float32))

    h, st = _fe1_call(x, fe_w1, fe_b1.reshape(1, H // 2))
    v, vlh = _fe2_call(h, st, fe_gamma.reshape(1, H // 2),
                       fe_beta.reshape(1, H // 2), fe_w2,
                       fe_b2.reshape(1, H), e_lin_w[0])

    for l in range(L - 1):
        agg = _edge_call(vlh, wfs[l], row2, col2)
        v, vlh = _upd_call(agg, v, v_w1[l], v_b1[l].reshape(1, H),
                           v_w2[l], v_b2[l].reshape(1, H), e_lin_w[l + 1])

    agg = _edge_call(vlh, wfs[L - 1], row2, col2)
    batch2 = batch.astype(jnp.int32).reshape(N, 1)
    u = _out_call(agg, v, v_w1[L - 1], v_b1[L - 1].reshape(1, H),
                  v_w2[L - 1], v_b2[L - 1].reshape(1, H),
                  u_w1, u_b1.reshape(1, H // 2), u_w2, u_b2.reshape(1, OUT),
                  batch2)
    return u


# speedup vs baseline: 2.1594x; 1.0639x over previous
import functools
import math

import jax
import jax.numpy as jnp
from jax import lax
from jax.experimental import pallas as pl
from jax.experimental.pallas import tpu as pltpu
from jax.experimental.pallas import tpu_sc as plsc

N = 50000
E = 800000
EP = 819200
L = 6
H = 64
NF = 64
NG = 50
OUT = 32
IN_DIM = 28
CUTOFF = 6.0
NGRAPH = 64
NPAD = 50008
LOG2 = math.log(2.0)

NB = 1000
NNODE_BLK = N // NB
EB = 1024
NEDGE_BLK = EP // EB

EPT = EP // 16
ECH = 128
NCH = EPT // ECH
AGGR = 50048
ROWS_PT = AGGR // 16
DRB = 136
NDR = ROWS_PT // DRB

D2_PW = EP // 32
D2_HALF = D2_PW // 2
D2_CH = 6400


def _sp(a):
    return jnp.maximum(a, 0.0) + jnp.log(1.0 + jnp.exp(-jnp.abs(a))) - LOG2


def _d2_body(posx, posy, posz, row1, col1, d2_out, posc, rowb, colb, d2b):
    c = lax.axis_index("c")
    s = lax.axis_index("s")
    wid = s * 2 + c
    base = wid * D2_PW
    post = (posx, posy, posz)
    for half in range(2):
        hbase = base + half * D2_HALF
        for coord in range(3):
            pltpu.sync_copy(post[coord], posc)
            for ch in range(2):
                cbase = hbase + ch * D2_CH
                pltpu.sync_copy(row1.at[pl.ds(cbase, D2_CH)], rowb)
                pltpu.sync_copy(col1.at[pl.ds(cbase, D2_CH)], colb)
                off = ch * D2_CH

                def go(j, _, off=off, first=(coord == 0)):
                    ri = rowb[pl.ds(j * 16, 16)]
                    ci = colb[pl.ds(j * 16, 16)]
                    a = plsc.load_gather(posc, [ri])
                    b = plsc.load_gather(posc, [ci])
                    dd = a - b
                    sl = pl.ds(off + j * 16, 16)
                    if first:
                        d2b[sl] = dd * dd
                    else:
                        d2b[sl] = d2b[sl] + dd * dd
                    return _

                lax.fori_loop(0, D2_CH // 16, go, 0)
        pltpu.sync_copy(d2b, d2_out.at[pl.ds(hbase, D2_HALF)])


def _d2_call(posx, posy, posz, row_p, col_p):
    f = pl.kernel(
        _d2_body,
        out_type=jax.ShapeDtypeStruct((EP,), jnp.float32),
        mesh=plsc.VectorSubcoreMesh(core_axis_name="c", subcore_axis_name="s"),
        scratch_types=[
            pltpu.VMEM((NPAD,), jnp.float32),
            pltpu.VMEM((D2_CH,), jnp.int32),
            pltpu.VMEM((D2_CH,), jnp.int32),
            pltpu.VMEM((D2_HALF,), jnp.float32),
        ],
        compiler_params=pltpu.CompilerParams(needs_layout_passes=False),
    )
    return f(posx, posy, posz, row_p, col_p)


SUP = 1024
NSUP = EPT // SUP
CPS = SUP // ECH


def _edge_body_full(vlh, wf, row2, col2, agg_out,
                    rowv, colv, rows, wfv, dbuf, agg_spm, gsem, wsem):
    c = lax.axis_index("c")
    s = lax.axis_index("s")

    @plsc.parallel_loop(0, DRB, step=1, unroll=8)
    def _z(m):
        dbuf[m, pl.ds(0, 16)] = jnp.zeros((16,), jnp.float32)
        dbuf[m, pl.ds(16, 16)] = jnp.zeros((16,), jnp.float32)

    def zs(k, _):
        pltpu.sync_copy(dbuf, agg_spm.at[pl.ds(s * ROWS_PT + k * DRB, DRB)])
        return 0

    lax.fori_loop(0, NDR, zs, 0)
    plsc.subcore_barrier()

    def sup(si, _):
        sb = s * EPT + si * SUP
        rb = s * (EPT // 128) + si * CPS
        pltpu.sync_copy(row2.at[pl.ds(rb, CPS)], rowv)
        pltpu.sync_copy(col2.at[pl.ds(rb, CPS)], colv)
        pltpu.async_copy(vlh.at[c].at[rowv.at[0]], rows.at[0], gsem)
        pltpu.async_copy(wf.at[c].at[pl.ds(sb, ECH)], wfv.at[0], wsem)
        for j in range(CPS):
            buf = j % 2
            e0 = sb + j * ECH
            if j + 1 < CPS:
                pltpu.async_copy(vlh.at[c].at[rowv.at[j + 1]],
                                 rows.at[1 - buf], gsem)
                pltpu.async_copy(wf.at[c].at[pl.ds(e0 + ECH, ECH)],
                                 wfv.at[1 - buf], wsem)
            pltpu.make_async_copy(wf.at[c].at[pl.ds(e0, ECH)],
                                  wfv.at[buf], wsem).wait()
            pltpu.make_async_copy(vlh.at[c].at[rowv.at[j]],
                                  rows.at[buf], gsem).wait()

            @plsc.parallel_loop(0, ECH, step=1, unroll=8)
            def _mul(m, buf=buf):
                rows[buf, m, pl.ds(0, 16)] = (rows[buf, m, pl.ds(0, 16)]
                                              * wfv[buf, m, pl.ds(0, 16)])
                rows[buf, m, pl.ds(16, 16)] = (rows[buf, m, pl.ds(16, 16)]
                                               * wfv[buf, m, pl.ds(16, 16)])

            pltpu.sync_copy(rows.at[buf], agg_spm.at[colv.at[j]], add=True)
        return 0

    lax.fori_loop(0, NSUP, sup, 0)
    plsc.subcore_barrier()

    def drain(k, _):
        rbase = s * ROWS_PT + k * DRB
        pltpu.sync_copy(agg_spm.at[pl.ds(rbase, DRB)], dbuf)
        pltpu.sync_copy(dbuf, agg_out.at[c].at[pl.ds(rbase, DRB)])
        return 0

    lax.fori_loop(0, NDR, drain, 0)


def _edge_call(vlh, wf, row2, col2):
    f = pl.kernel(
        _edge_body_full,
        out_type=jax.ShapeDtypeStruct((2, AGGR, OUT), jnp.float32),
        mesh=plsc.VectorSubcoreMesh(core_axis_name="c", subcore_axis_name="s"),
        scratch_types=[
            pltpu.VMEM((CPS, 128), jnp.int32),
            pltpu.VMEM((CPS, 128), jnp.int32),
            pltpu.VMEM((2, ECH, OUT), jnp.float32),
            pltpu.VMEM((2, ECH, OUT), jnp.float32),
            pltpu.VMEM((DRB, OUT), jnp.float32),
            pltpu.VMEM_SHARED((AGGR, OUT), jnp.float32),
            pltpu.SemaphoreType.DMA,
            pltpu.SemaphoreType.DMA,
        ],
        compiler_params=pltpu.CompilerParams(needs_layout_passes=False,
                                             use_tc_tiling_on_sc=False),
    )
    return f(vlh, wf, row2, col2)


def _wf_body(d2_ref, w1a_ref, b1a_ref, w2_ref, b2_ref, eye_ref, *out_refs):
    step = CUTOFF / (NG - 1)
    coeff = -0.5 / (step * step)
    d2 = d2_ref[...]
    dist = jnp.sqrt(d2)
    cutf = 0.5 * (jnp.cos(dist * (math.pi / CUTOFF)) + 1.0)
    offs = (lax.broadcasted_iota(jnp.int32, (NG, 128), 0).astype(jnp.float32)
            * step)
    eye = eye_ref[...]
    cols = []
    for g in range(8):
        dr = dist[g:g + 1, :]
        et = jnp.exp(coeff * (dr - offs) ** 2)
        etc = jnp.concatenate([et, cutf[g:g + 1, :]], axis=0)
        cols.append(lax.dot_general(eye, etc, (((1,), (1,)), ((), ())),
                                    preferred_element_type=jnp.float32))
    embc = jnp.concatenate(cols, axis=0)
    cc = embc[:, NG:NG + 1]
    h1 = _sp(jnp.dot(embc, w1a_ref[...],
                     preferred_element_type=jnp.float32) + b1a_ref[...])
    for l in range(L):
        wfl = jnp.dot(h1[:, l * NF:(l + 1) * NF], w2_ref[l],
                      preferred_element_type=jnp.float32) + b2_ref[l]
        wfl = wfl * cc
        out_refs[l][0] = wfl[:, :OUT]
        out_refs[l][1] = wfl[:, OUT:]


def _wf_call(d2m, w1a, b1a, w2, b2c, eye):
    return pl.pallas_call(
        _wf_body,
        grid=(NEDGE_BLK,),
        in_specs=[
            pl.BlockSpec((8, 128), lambda i: (i, 0)),
            pl.BlockSpec((NG + 1, L * NF), lambda i: (0, 0)),
            pl.BlockSpec((1, L * NF), lambda i: (0, 0)),
            pl.BlockSpec((L, NF, NF), lambda i: (0, 0, 0)),
            pl.BlockSpec((L, 1, NF), lambda i: (0, 0, 0)),
            pl.BlockSpec((128, 128), lambda i: (0, 0)),
        ],
        out_specs=[pl.BlockSpec((2, EB, OUT), lambda i: (0, i, 0))
                   for _ in range(L)],
        out_shape=[jax.ShapeDtypeStruct((2, EP, OUT), jnp.float32)
                   for _ in range(L)],
    )(d2m, w1a, b1a, w2, b2c, eye)


def _fe1_body(x_ref, w_ref, b_ref, h_ref, st_ref, acc_ref):
    i = pl.program_id(0)
    h = jnp.dot(x_ref[...], w_ref[...], preferred_element_type=jnp.float32) \
        + b_ref[...]
    h_ref[...] = h
    s1 = jnp.sum(h, axis=0, keepdims=True)
    s2 = jnp.sum(h * h, axis=0, keepdims=True)
    st = jnp.concatenate([s1, s2, jnp.zeros((6, H // 2), jnp.float32)], axis=0)

    @pl.when(i == 0)
    def _():
        acc_ref[...] = jnp.zeros_like(acc_ref)

    acc_ref[...] += st

    @pl.when(i == NNODE_BLK - 1)
    def _():
        st_ref[...] = acc_ref[...]


def _fe1_call(x, w1, b1):
    return pl.pallas_call(
        _fe1_body,
        grid=(NNODE_BLK,),
        in_specs=[
            pl.BlockSpec((NB, IN_DIM), lambda i: (i, 0)),
            pl.BlockSpec((IN_DIM, H // 2), lambda i: (0, 0)),
            pl.BlockSpec((1, H // 2), lambda i: (0, 0)),
        ],
        out_specs=[
            pl.BlockSpec((NB, H // 2), lambda i: (i, 0)),
            pl.BlockSpec((8, H // 2), lambda i: (0, 0)),
        ],
        out_shape=[
            jax.ShapeDtypeStruct((N, H // 2), jnp.float32),
            jax.ShapeDtypeStruct((8, H // 2), jnp.float32),
        ],
        scratch_shapes=[pltpu.VMEM((8, H // 2), jnp.float32)],
    )(x, w1, b1)


def _fe2_body(h_ref, st_ref, g_ref, be_ref, w2_ref, b2_ref, elw_ref,
              v_ref, vlh_ref):
    st = st_ref[...]
    mean = st[0:1] / N
    var = st[1:2] / N - mean * mean
    inv = lax.rsqrt(var + 1e-5)
    hn = (h_ref[...] - mean) * inv * g_ref[...] + be_ref[...]
    hn = jnp.maximum(hn, 0.0)
    v = jnp.maximum(
        jnp.dot(hn, w2_ref[...], preferred_element_type=jnp.float32)
        + b2_ref[...], 0.0)
    v_ref[...] = v
    vl = jnp.dot(v, elw_ref[...], preferred_element_type=jnp.float32)
    vlh_ref[0] = vl[:, :OUT]
    vlh_ref[1] = vl[:, OUT:]


def _fe2_call(h, st, g, be, w2, b2, elw):
    return pl.pallas_call(
        _fe2_body,
        grid=(NNODE_BLK,),
        in_specs=[
            pl.BlockSpec((NB, H // 2), lambda i: (i, 0)),
            pl.BlockSpec((8, H // 2), lambda i: (0, 0)),
            pl.BlockSpec((1, H // 2), lambda i: (0, 0)),
            pl.BlockSpec((1, H // 2), lambda i: (0, 0)),
            pl.BlockSpec((H // 2, H), lambda i: (0, 0)),
            pl.BlockSpec((1, H), lambda i: (0, 0)),
            pl.BlockSpec((H, NF), lambda i: (0, 0)),
        ],
        out_specs=[
            pl.BlockSpec((NB, H), lambda i: (i, 0)),
            pl.BlockSpec((2, NB, OUT), lambda i: (0, i, 0)),
        ],
        out_shape=[
            jax.ShapeDtypeStruct((N, H), jnp.float32),
            jax.ShapeDtypeStruct((2, N, OUT), jnp.float32),
        ],
    )(h, st, g, be, w2, b2, elw)


def _upd_body(agg_ref, v_ref, w1_ref, b1_ref, w2_ref, b2_ref, elw_ref,
              vn_ref, vlh_ref):
    cat = jnp.concatenate([agg_ref[0], agg_ref[1]], axis=1)
    m = jnp.dot(_sp(jnp.dot(cat, w1_ref[...],
                            preferred_element_type=jnp.float32) + b1_ref[...]),
                w2_ref[...], preferred_element_type=jnp.float32) + b2_ref[...]
    vn = v_ref[...] + m
    vn_ref[...] = vn
    vl = jnp.dot(vn, elw_ref[...], preferred_element_type=jnp.float32)
    vlh_ref[0] = vl[:, :OUT]
    vlh_ref[1] = vl[:, OUT:]


def _upd_call(agg, v, w1, b1, w2, b2, elw):
    return pl.pallas_call(
        _upd_body,
        grid=(NNODE_BLK,),
        in_specs=[
            pl.BlockSpec((2, NB, OUT), lambda i: (0, i, 0)),
            pl.BlockSpec((NB, H), lambda i: (i, 0)),
            pl.BlockSpec((NF, H), lambda i: (0, 0)),
            pl.BlockSpec((1, H), lambda i: (0, 0)),
            pl.BlockSpec((H, H), lambda i: (0, 0)),
            pl.BlockSpec((1, H), lambda i: (0, 0)),
            pl.BlockSpec((H, NF), lambda i: (0, 0)),
        ],
        out_specs=[
            pl.BlockSpec((NB, H), lambda i: (i, 0)),
            pl.BlockSpec((2, NB, OUT), lambda i: (0, i, 0)),
        ],
        out_shape=[
            jax.ShapeDtypeStruct((N, H), jnp.float32),
            jax.ShapeDtypeStruct((2, N, OUT), jnp.float32),
        ],
    )(agg, v, w1, b1, w2, b2, elw)


def _out_body(agg_ref, v_ref, w1_ref, b1_ref, w2_ref, b2_ref,
              uw1_ref, ub1_ref, uw2_ref, ub2_ref, batch_ref,
              u_ref, acc_ref):
    i = pl.program_id(0)
    cat = jnp.concatenate([agg_ref[0], agg_ref[1]], axis=1)
    m = jnp.dot(_sp(jnp.dot(cat, w1_ref[...],
                            preferred_element_type=jnp.float32) + b1_ref[...]),
                w2_ref[...], preferred_element_type=jnp.float32) + b2_ref[...]
    vn = v_ref[...] + m
    hu = jnp.dot(_sp(jnp.dot(vn, uw1_ref[...],
                             preferred_element_type=jnp.float32) + ub1_ref[...]),
                 uw2_ref[...], preferred_element_type=jnp.float32) + ub2_ref[...]
    gid = lax.broadcasted_iota(jnp.int32, (NB, NGRAPH), 1)
    oh = (gid == batch_ref[...]).astype(jnp.float32)
    part = lax.dot_general(oh, hu, (((0,), (0,)), ((), ())),
                           preferred_element_type=jnp.float32)

    @pl.when(i == 0)
    def _():
        acc_ref[...] = jnp.zeros_like(acc_ref)

    acc_ref[...] += part

    @pl.when(i == NNODE_BLK - 1)
    def _():
        u_ref[...] = acc_ref[...]


def _out_call(agg, v, w1, b1, w2, b2, uw1, ub1, uw2, ub2, batch2):
    return pl.pallas_call(
        _out_body,
        grid=(NNODE_BLK,),
        in_specs=[
            pl.BlockSpec((2, NB, OUT), lambda i: (0, i, 0)),
            pl.BlockSpec((NB, H), lambda i: (i, 0)),
            pl.BlockSpec((NF, H), lambda i: (0, 0)),
            pl.BlockSpec((1, H), lambda i: (0, 0)),
            pl.BlockSpec((H, H), lambda i: (0, 0)),
            pl.BlockSpec((1, H), lambda i: (0, 0)),
            pl.BlockSpec((H, H // 2), lambda i: (0, 0)),
            pl.BlockSpec((1, H // 2), lambda i: (0, 0)),
            pl.BlockSpec((H // 2, OUT), lambda i: (0, 0)),
            pl.BlockSpec((1, OUT), lambda i: (0, 0)),
            pl.BlockSpec((NB, 1), lambda i: (i, 0)),
        ],
        out_specs=pl.BlockSpec((NGRAPH, OUT), lambda i: (0, 0)),
        out_shape=jax.ShapeDtypeStruct((NGRAPH, OUT), jnp.float32),
        scratch_shapes=[pltpu.VMEM((NGRAPH, OUT), jnp.float32)],
    )(agg, v, w1, b1, w2, b2, uw1, ub1, uw2, ub2, batch2)


def kernel(x, x_one_hot, pos, batch, edge_index,
           fe_w1, fe_b1, fe_gamma, fe_beta, fe_w2, fe_b2,
           e_lin_w, e_mlp_w1, e_mlp_b1, e_mlp_w2, e_mlp_b2,
           v_w1, v_b1, v_w2, v_b2, u_w1, u_b1, u_w2, u_b2):
    row = edge_index[0].astype(jnp.int32)
    col = edge_index[1].astype(jnp.int32)
    row_p = jnp.concatenate([row, jnp.zeros((EP - E,), jnp.int32)])
    col_p = jnp.concatenate([col, jnp.full((EP - E,), N, jnp.int32)])
    row2 = row_p.reshape(EP // 128, 128)
    col2 = col_p.reshape(EP // 128, 128)

    zpad = jnp.zeros((NPAD - N,), jnp.float32)
    posx = jnp.concatenate([pos[:, 0], zpad])
    posy = jnp.concatenate([pos[:, 1], zpad])
    posz = jnp.concatenate([pos[:, 2], zpad])

    d2 = _d2_call(posx, posy, posz, row_p, col_p)
    d2m = d2.reshape(EP // 128, 128)

    w1a = jnp.concatenate([jnp.moveaxis(e_mlp_w1, 0, 1).reshape(NG, L * NF),
                           jnp.zeros((1, L * NF), jnp.float32)], axis=0)
    wfs = _wf_call(d2m, w1a, e_mlp_b1.reshape(1, L * NF),
                   e_mlp_w2, e_mlp_b2.reshape(L, 1, NF),
                   jnp.eye(128, dtype=jnp.float32))

    h, st = _fe1_call(x, fe_w1, fe_b1.reshape(1, H // 2))
    v, vlh = _fe2_call(h, st, fe_gamma.reshape(1, H // 2),
                       fe_beta.reshape(1, H // 2), fe_w2,
                       fe_b2.reshape(1, H), e_lin_w[0])

    for l in range(L - 1):
        agg = _edge_call(vlh, wfs[l], row2, col2)
        v, vlh = _upd_call(agg, v, v_w1[l], v_b1[l].reshape(1, H),
                           v_w2[l], v_b2[l].reshape(1, H), e_lin_w[l + 1])

    agg = _edge_call(vlh, wfs[L - 1], row2, col2)
    batch2 = batch.astype(jnp.int32).reshape(N, 1)
    u = _out_call(agg, v, v_w1[L - 1], v_b1[L - 1].reshape(1, H),
                  v_w2[L - 1], v_b2[L - 1].reshape(1, H),
                  u_w1, u_b1.reshape(1, H // 2), u_w2, u_b2.reshape(1, OUT),
                  batch2)
    return u
